# baseline (speedup 1.0000x reference)
import functools

import jax
import jax.numpy as jnp
from jax import lax
from jax.experimental import pallas as pl
from jax.experimental.pallas import tpu as pltpu
from jax.experimental.pallas import tpu_sc as plsc

B, S, D = 4, 4096, 2048
K = 512
NC, NS = 2, 16
NW = NC * NS
ROWS_TOTAL = B * K
ROWS_PER_W = ROWS_TOTAL // NW
CHUNK = 16
NCHUNK = ROWS_PER_W // CHUNK


def _sumsq_body(key_ref, out_ref):
    x = key_ref[...]
    acc = x[:, :, 0:128] * x[:, :, 0:128]
    for c in range(1, D // 128):
        xc = x[:, :, c * 128:(c + 1) * 128]
        acc = acc + xc * xc
    u = acc[:, :, 0:8]
    for k in range(1, 16):
        u = u + acc[:, :, 8 * k:8 * k + 8]
    v1 = u[:, :, :4] + u[:, :, 4:]
    v2 = v1[:, :, :2] + v1[:, :, 2:]
    out_ref[...] = jnp.sqrt(v2[:, :, 0] + v2[:, :, 1])


S_BLK = 256


def _importance(key):
    return pl.pallas_call(
        _sumsq_body,
        grid=(S // S_BLK,),
        in_specs=[pl.BlockSpec((B, S_BLK, D), lambda i: (0, i, 0))],
        out_specs=pl.BlockSpec((B, S_BLK), lambda i: (0, i)),
        out_shape=jax.ShapeDtypeStruct((B, S), jnp.float32),
    )(key)


I_BLK = 512


def _rank_body(vi_ref, vj_ref, out_ref):
    b = pl.program_id(0)
    c = pl.program_id(1)
    vi_row = vi_ref[0]
    vj_row = vj_ref[0]

    ii = lax.broadcasted_iota(jnp.int32, (I_BLK, I_BLK), 0)
    jj = lax.broadcasted_iota(jnp.int32, (I_BLK, I_BLK), 1)
    vi_bc = jnp.broadcast_to(vi_row, (I_BLK, I_BLK))
    vi_col = jnp.sum(jnp.where(ii == jj, vi_bc, 0.0), axis=1, keepdims=True)

    i_col = c * I_BLK + lax.broadcasted_iota(jnp.int32, (I_BLK, 1), 0)
    j_row = lax.broadcasted_iota(jnp.int32, (1, S), 1)
    beats = (vj_row > vi_col) | ((vj_row == vi_col) & (j_row < i_col))
    rank_col = jnp.sum(beats.astype(jnp.int32), axis=1, keepdims=True)

    r_row = lax.broadcasted_iota(jnp.int32, (1, K), 1)
    sel = rank_col == r_row
    flat_col = jnp.broadcast_to(i_col + b * S, (I_BLK, K))
    contrib = jnp.sum(jnp.where(sel, flat_col, 0), axis=0, keepdims=True)

    @pl.when(c == 0)
    def _():
        out_ref[0] = contrib

    @pl.when(c > 0)
    def _():
        out_ref[0] = out_ref[0] + contrib


def _topk_flat_idx(imp3):
    return pl.pallas_call(
        _rank_body,
        grid=(B, S // I_BLK),
        in_specs=[
            pl.BlockSpec((1, 1, I_BLK), lambda b, c: (b, 0, c)),
            pl.BlockSpec((1, 1, S), lambda b, c: (b, 0, 0)),
        ],
        out_specs=pl.BlockSpec((1, 1, K), lambda b, c: (b, 0, 0)),
        out_shape=jax.ShapeDtypeStruct((B, 1, K), jnp.int32),
    )(imp3, imp3)


def _gather_body(key_hbm, val_hbm, idx_hbm, out_hbm,
                 idx_v, kbuf, vbuf, semk, semv):
    cid = lax.axis_index("c")
    sid = lax.axis_index("s")
    wid = sid * NC + cid
    base = wid * ROWS_PER_W
    for ch in range(NCHUNK):
        gb = base + ch * CHUNK
        pltpu.sync_copy(idx_hbm.at[pl.ds(gb, CHUNK)], idx_v)
        ck = pltpu.async_copy(key_hbm.at[idx_v], kbuf, semk)
        cv = pltpu.async_copy(val_hbm.at[idx_v], vbuf, semv)
        lanes = lax.iota(jnp.int32, CHUNK)
        kodst = (gb + lanes) * 2
        vodst = kodst + 1
        ck.wait()
        cv.wait()
        wk = pltpu.async_copy(kbuf, out_hbm.at[kodst], semk)
        wv = pltpu.async_copy(vbuf, out_hbm.at[vodst], semv)
        wk.wait()
        wv.wait()


@functools.cache
def _sc_gather():
    return pl.kernel(
        _gather_body,
        out_type=jax.ShapeDtypeStruct((2 * ROWS_TOTAL, D), jnp.float32),
        mesh=plsc.VectorSubcoreMesh(core_axis_name="c", subcore_axis_name="s"),
        scratch_types=[
            pltpu.VMEM((CHUNK,), jnp.int32),
            pltpu.VMEM((CHUNK, D), jnp.float32),
            pltpu.VMEM((CHUNK, D), jnp.float32),
            pltpu.SemaphoreType.DMA,
            pltpu.SemaphoreType.DMA,
        ],
    )


def kernel(key, value):
    imp = _importance(key)
    idx = _topk_flat_idx(imp.reshape(B, 1, S))
    out2 = _sc_gather()(key.reshape(B * S, D), value.reshape(B * S, D),
                        idx.reshape(ROWS_TOTAL))
    return out2.reshape(B, K, 2 * D)


# speedup vs baseline: 1.2188x; 1.2188x over previous
import functools

import jax
import jax.numpy as jnp
from jax import lax
from jax.experimental import pallas as pl
from jax.experimental.pallas import tpu as pltpu
from jax.experimental.pallas import tpu_sc as plsc

B, S, D = 4, 4096, 2048
K = 512
NC, NS = 2, 16
NW = NC * NS
ROWS_TOTAL = B * K
ROWS_PER_W = ROWS_TOTAL // NW
CHUNK = 16
NCHUNK = ROWS_PER_W // CHUNK


def _sumsq_body(key_ref, out_ref):
    x = key_ref[...]
    acc = x[:, :, 0:128] * x[:, :, 0:128]
    for c in range(1, D // 128):
        xc = x[:, :, c * 128:(c + 1) * 128]
        acc = acc + xc * xc
    u = acc[:, :, 0:8]
    for k in range(1, 16):
        u = u + acc[:, :, 8 * k:8 * k + 8]
    v1 = u[:, :, :4] + u[:, :, 4:]
    v2 = v1[:, :, :2] + v1[:, :, 2:]
    out_ref[...] = jnp.sqrt(v2[:, :, 0] + v2[:, :, 1])


S_BLK = 256


def _importance(key):
    return pl.pallas_call(
        _sumsq_body,
        grid=(S // S_BLK,),
        in_specs=[pl.BlockSpec((B, S_BLK, D), lambda i: (0, i, 0))],
        out_specs=pl.BlockSpec((B, S_BLK), lambda i: (0, i)),
        out_shape=jax.ShapeDtypeStruct((B, S), jnp.float32),
    )(key)


_PC = 512


def _select_body(imp_ref, out_ref):
    b = pl.program_id(0)
    v_row = imp_ref[0]
    bits = lax.bitcast_convert_type(v_row, jnp.int32)
    j_row = lax.broadcasted_iota(jnp.int32, (1, S), 1)

    t = jnp.zeros((1, 1), jnp.int32)
    for bit in range(30, -1, -1):
        cand = t | (1 << bit)
        cnt = jnp.sum((bits >= cand).astype(jnp.int32), axis=1, keepdims=True)
        t = jnp.where(cnt >= K, cand, t)
    strict = bits > t
    tie = bits == t
    m = jnp.sum(strict.astype(jnp.int32), axis=1, keepdims=True)
    need = K - m

    xm = jnp.zeros((1, 1), jnp.int32)
    for bit in range(11, -1, -1):
        cand = xm | (1 << bit)
        cnt = jnp.sum((tie & (j_row < cand)).astype(jnp.int32),
                      axis=1, keepdims=True)
        xm = jnp.where(cnt < need, cand, xm)
    sel_mask = strict | (tie & (j_row < (xm + 1)))
    candi = sel_mask.astype(jnp.int32)

    p = candi
    sh = 1
    while sh < S:
        shifted = jnp.concatenate(
            [jnp.zeros((1, sh), jnp.int32), p[:, :S - sh]], axis=1)
        p = p + shifted
        sh *= 2
    p_row = p - candi

    r_col = lax.broadcasted_iota(jnp.int32, (K, 1), 0)
    g = (p_row == r_col) & sel_mask
    bits_bc = jnp.broadcast_to(bits, (K, S))
    j_bc = jnp.broadcast_to(j_row, (K, S))
    bits_col = jnp.sum(jnp.where(g, bits_bc, 0), axis=1, keepdims=True)
    idx_col = jnp.sum(jnp.where(g, j_bc, 0), axis=1, keepdims=True)

    ii = lax.broadcasted_iota(jnp.int32, (K, K), 0)
    jj = lax.broadcasted_iota(jnp.int32, (K, K), 1)
    eye = ii == jj
    bits_row = jnp.sum(jnp.where(eye, jnp.broadcast_to(bits_col, (K, K)), 0),
                       axis=0, keepdims=True)
    idx_row = jnp.sum(jnp.where(eye, jnp.broadcast_to(idx_col, (K, K)), 0),
                      axis=0, keepdims=True)

    beats = (bits_row > bits_col) | ((bits_row == bits_col) &
                                     (idx_row < idx_col))
    rank_col = jnp.sum(beats.astype(jnp.int32), axis=1, keepdims=True)

    r_rowi = lax.broadcasted_iota(jnp.int32, (1, K), 1)
    sel = rank_col == r_rowi
    flat_col = jnp.broadcast_to(idx_col + b * S, (K, K))
    out_ref[0] = jnp.sum(jnp.where(sel, flat_col, 0), axis=0, keepdims=True)


def _topk_flat_idx(imp3):
    return pl.pallas_call(
        _select_body,
        grid=(B,),
        in_specs=[pl.BlockSpec((1, 1, S), lambda b: (b, 0, 0))],
        out_specs=pl.BlockSpec((1, 1, K), lambda b: (b, 0, 0)),
        out_shape=jax.ShapeDtypeStruct((B, 1, K), jnp.int32),
    )(imp3)


def _gather_body(key_hbm, val_hbm, idx_hbm, out_hbm,
                 idx_v, kbuf, vbuf, semk, semv):
    cid = lax.axis_index("c")
    sid = lax.axis_index("s")
    wid = sid * NC + cid
    base = wid * ROWS_PER_W
    for ch in range(NCHUNK):
        gb = base + ch * CHUNK
        pltpu.sync_copy(idx_hbm.at[pl.ds(gb, CHUNK)], idx_v)
        idx_v[...] = jnp.clip(idx_v[...], 0, B * S - 1)
        ck = pltpu.async_copy(key_hbm.at[idx_v], kbuf, semk)
        cv = pltpu.async_copy(val_hbm.at[idx_v], vbuf, semv)
        lanes = lax.iota(jnp.int32, CHUNK)
        kodst = (gb + lanes) * 2
        vodst = kodst + 1
        ck.wait()
        cv.wait()
        wk = pltpu.async_copy(kbuf, out_hbm.at[kodst], semk)
        wv = pltpu.async_copy(vbuf, out_hbm.at[vodst], semv)
        wk.wait()
        wv.wait()


@functools.cache
def _sc_gather():
    return pl.kernel(
        _gather_body,
        out_type=jax.ShapeDtypeStruct((2 * ROWS_TOTAL, D), jnp.float32),
        mesh=plsc.VectorSubcoreMesh(core_axis_name="c", subcore_axis_name="s"),
        scratch_types=[
            pltpu.VMEM((CHUNK,), jnp.int32),
            pltpu.VMEM((CHUNK, D), jnp.float32),
            pltpu.VMEM((CHUNK, D), jnp.float32),
            pltpu.SemaphoreType.DMA,
            pltpu.SemaphoreType.DMA,
        ],
    )


def kernel(key, value):
    imp = _importance(key)
    idx = _topk_flat_idx(imp.reshape(B, 1, S))
    out2 = _sc_gather()(key.reshape(B * S, D), value.reshape(B * S, D),
                        idx.reshape(ROWS_TOTAL))
    return out2.reshape(B, K, 2 * D)


# speedup vs baseline: 1.3633x; 1.1185x over previous
import functools

import jax
import jax.numpy as jnp
from jax import lax
from jax.experimental import pallas as pl
from jax.experimental.pallas import tpu as pltpu
from jax.experimental.pallas import tpu_sc as plsc

B, S, D = 4, 4096, 2048
K = 512
NC, NS = 2, 16
NW = NC * NS
ROWS_TOTAL = B * K
ROWS_PER_W = ROWS_TOTAL // NW
CHUNK = 16
NCHUNK = ROWS_PER_W // CHUNK


def _sumsq_body(key_ref, out_ref):
    x = key_ref[...]
    acc = x[:, :, 0:128] * x[:, :, 0:128]
    for c in range(1, D // 128):
        xc = x[:, :, c * 128:(c + 1) * 128]
        acc = acc + xc * xc
    u = acc[:, :, 0:8]
    for k in range(1, 16):
        u = u + acc[:, :, 8 * k:8 * k + 8]
    v1 = u[:, :, :4] + u[:, :, 4:]
    v2 = v1[:, :, :2] + v1[:, :, 2:]
    out_ref[...] = jnp.sqrt(v2[:, :, 0] + v2[:, :, 1])


S_BLK = 256


def _importance(key):
    return pl.pallas_call(
        _sumsq_body,
        grid=(S // S_BLK,),
        in_specs=[pl.BlockSpec((B, S_BLK, D), lambda i: (0, i, 0))],
        out_specs=pl.BlockSpec((B, S_BLK), lambda i: (0, i)),
        out_shape=jax.ShapeDtypeStruct((B, S), jnp.float32),
    )(key)


_PC = 512


def _select_body(imp_ref, out_ref):
    b = pl.program_id(0)
    v_row = imp_ref[0]
    bits = lax.bitcast_convert_type(v_row, jnp.int32)
    j_row = lax.broadcasted_iota(jnp.int32, (1, S), 1)

    t = jnp.zeros((1, 1), jnp.int32)
    for bit in range(30, -1, -1):
        cand = t | (1 << bit)
        cnt = jnp.sum((bits >= cand).astype(jnp.int32), axis=1, keepdims=True)
        t = jnp.where(cnt >= K, cand, t)
    strict = bits > t
    tie = bits == t
    m = jnp.sum(strict.astype(jnp.int32), axis=1, keepdims=True)
    need = K - m

    xm = jnp.zeros((1, 1), jnp.int32)
    for bit in range(11, -1, -1):
        cand = xm | (1 << bit)
        cnt = jnp.sum((tie & (j_row < cand)).astype(jnp.int32),
                      axis=1, keepdims=True)
        xm = jnp.where(cnt < need, cand, xm)
    sel_mask = strict | (tie & (j_row < (xm + 1)))
    candi = sel_mask.astype(jnp.int32)

    p = candi
    sh = 1
    while sh < S:
        shifted = jnp.concatenate(
            [jnp.zeros((1, sh), jnp.int32), p[:, :S - sh]], axis=1)
        p = p + shifted
        sh *= 2
    p_row = p - candi

    r_col = lax.broadcasted_iota(jnp.int32, (K, 1), 0)
    g = (p_row == r_col) & sel_mask
    bits_bc = jnp.broadcast_to(bits, (K, S))
    j_bc = jnp.broadcast_to(j_row, (K, S))
    bits_col = jnp.sum(jnp.where(g, bits_bc, 0), axis=1, keepdims=True)
    idx_col = jnp.sum(jnp.where(g, j_bc, 0), axis=1, keepdims=True)

    ii = lax.broadcasted_iota(jnp.int32, (K, K), 0)
    jj = lax.broadcasted_iota(jnp.int32, (K, K), 1)
    eye = ii == jj
    bits_row = jnp.sum(jnp.where(eye, jnp.broadcast_to(bits_col, (K, K)), 0),
                       axis=0, keepdims=True)
    idx_row = jnp.sum(jnp.where(eye, jnp.broadcast_to(idx_col, (K, K)), 0),
                      axis=0, keepdims=True)

    beats = (bits_row > bits_col) | ((bits_row == bits_col) &
                                     (idx_row < idx_col))
    rank_col = jnp.sum(beats.astype(jnp.int32), axis=1, keepdims=True)

    r_rowi = lax.broadcasted_iota(jnp.int32, (1, K), 1)
    sel = rank_col == r_rowi
    flat_col = jnp.broadcast_to(idx_col + b * S, (K, K))
    out_ref[0] = jnp.sum(jnp.where(sel, flat_col, 0), axis=0, keepdims=True)


def _topk_flat_idx(imp3):
    return pl.pallas_call(
        _select_body,
        grid=(B,),
        in_specs=[pl.BlockSpec((1, 1, S), lambda b: (b, 0, 0))],
        out_specs=pl.BlockSpec((1, 1, K), lambda b: (b, 0, 0)),
        out_shape=jax.ShapeDtypeStruct((B, 1, K), jnp.int32),
    )(imp3)


def _merged_body(key_ref, idx_ref, imp_s):
    i = pl.program_id(0)
    x = key_ref[...]
    acc = x[:, :, 0:128] * x[:, :, 0:128]
    for c in range(1, D // 128):
        xc = x[:, :, c * 128:(c + 1) * 128]
        acc = acc + xc * xc
    u = acc[:, :, 0:8]
    for k in range(1, 16):
        u = u + acc[:, :, 8 * k:8 * k + 8]
    v1 = u[:, :, :4] + u[:, :, 4:]
    v2 = v1[:, :, :2] + v1[:, :, 2:]
    imp_s[:, pl.ds(i * S_BLK, S_BLK)] = jnp.sqrt(v2[:, :, 0] + v2[:, :, 1])

    @pl.when(i == S // S_BLK - 1)
    def _epilogue():
        bits4 = lax.bitcast_convert_type(imp_s[...], jnp.int32)
        j_row4 = lax.broadcasted_iota(jnp.int32, (B, S), 1)

        t = jnp.zeros((B, 1), jnp.int32)
        for bit in range(30, -1, -1):
            cand = t | (1 << bit)
            cnt = jnp.sum((bits4 >= cand).astype(jnp.int32),
                          axis=1, keepdims=True)
            t = jnp.where(cnt >= K, cand, t)
        strict = bits4 > t
        tie = bits4 == t
        m = jnp.sum(strict.astype(jnp.int32), axis=1, keepdims=True)
        need = K - m

        xm = jnp.zeros((B, 1), jnp.int32)
        for bit in range(11, -1, -1):
            cand = xm | (1 << bit)
            cnt = jnp.sum((tie & (j_row4 < cand)).astype(jnp.int32),
                          axis=1, keepdims=True)
            xm = jnp.where(cnt < need, cand, xm)
        sel4 = strict | (tie & (j_row4 < (xm + 1)))
        candi = sel4.astype(jnp.int32)

        p = candi
        sh = 1
        while sh < S:
            shifted = jnp.concatenate(
                [jnp.zeros((B, sh), jnp.int32), p[:, :S - sh]], axis=1)
            p = p + shifted
            sh *= 2
        p4 = p - candi

        r_col = lax.broadcasted_iota(jnp.int32, (K, 1), 0)
        ii = lax.broadcasted_iota(jnp.int32, (K, K), 0)
        jj = lax.broadcasted_iota(jnp.int32, (K, K), 1)
        eye = ii == jj
        r_rowi = lax.broadcasted_iota(jnp.int32, (1, K), 1)
        j_row = j_row4[0:1, :]

        for b in range(B):
            bits = bits4[b:b + 1, :]
            g = (p4[b:b + 1, :] == r_col) & sel4[b:b + 1, :]
            bits_bc = jnp.broadcast_to(bits, (K, S))
            j_bc = jnp.broadcast_to(j_row, (K, S))
            bits_col = jnp.sum(jnp.where(g, bits_bc, 0), axis=1, keepdims=True)
            idx_col = jnp.sum(jnp.where(g, j_bc, 0), axis=1, keepdims=True)

            bits_row = jnp.sum(
                jnp.where(eye, jnp.broadcast_to(bits_col, (K, K)), 0),
                axis=0, keepdims=True)
            idx_row = jnp.sum(
                jnp.where(eye, jnp.broadcast_to(idx_col, (K, K)), 0),
                axis=0, keepdims=True)

            beats = (bits_row > bits_col) | ((bits_row == bits_col) &
                                             (idx_row < idx_col))
            rank_col = jnp.sum(beats.astype(jnp.int32), axis=1, keepdims=True)

            sel = rank_col == r_rowi
            flat_col = jnp.broadcast_to(idx_col + b * S, (K, K))
            idx_ref[b:b + 1, :] = jnp.sum(jnp.where(sel, flat_col, 0),
                                          axis=0, keepdims=True)


def _imp_topk(key):
    return pl.pallas_call(
        _merged_body,
        grid=(S // S_BLK,),
        in_specs=[pl.BlockSpec((B, S_BLK, D), lambda i: (0, i, 0))],
        out_specs=pl.BlockSpec((B, K), lambda i: (0, 0)),
        out_shape=jax.ShapeDtypeStruct((B, K), jnp.int32),
        scratch_shapes=[pltpu.VMEM((B, S), jnp.float32)],
    )(key)


def _gather_body(key_hbm, val_hbm, idx_hbm, out_hbm,
                 idx_v, kbuf, vbuf, semk, semv):
    cid = lax.axis_index("c")
    sid = lax.axis_index("s")
    wid = sid * NC + cid
    base = wid * ROWS_PER_W
    for ch in range(NCHUNK):
        gb = base + ch * CHUNK
        pltpu.sync_copy(idx_hbm.at[pl.ds(gb, CHUNK)], idx_v)
        idx_v[...] = jnp.clip(idx_v[...], 0, B * S - 1)
        ck = pltpu.async_copy(key_hbm.at[idx_v], kbuf, semk)
        cv = pltpu.async_copy(val_hbm.at[idx_v], vbuf, semv)
        lanes = lax.iota(jnp.int32, CHUNK)
        kodst = (gb + lanes) * 2
        vodst = kodst + 1
        ck.wait()
        cv.wait()
        wk = pltpu.async_copy(kbuf, out_hbm.at[kodst], semk)
        wv = pltpu.async_copy(vbuf, out_hbm.at[vodst], semv)
        wk.wait()
        wv.wait()


@functools.cache
def _sc_gather():
    return pl.kernel(
        _gather_body,
        out_type=jax.ShapeDtypeStruct((2 * ROWS_TOTAL, D), jnp.float32),
        mesh=plsc.VectorSubcoreMesh(core_axis_name="c", subcore_axis_name="s"),
        scratch_types=[
            pltpu.VMEM((CHUNK,), jnp.int32),
            pltpu.VMEM((CHUNK, D), jnp.float32),
            pltpu.VMEM((CHUNK, D), jnp.float32),
            pltpu.SemaphoreType.DMA,
            pltpu.SemaphoreType.DMA,
        ],
    )


def kernel(key, value):
    idx = _imp_topk(key)
    out2 = _sc_gather()(key.reshape(B * S, D), value.reshape(B * S, D),
                        idx.reshape(ROWS_TOTAL))
    return out2.reshape(B, K, 2 * D)


# speedup vs baseline: 1.3960x; 1.0240x over previous
import functools

import jax
import jax.numpy as jnp
from jax import lax
from jax.experimental import pallas as pl
from jax.experimental.pallas import tpu as pltpu
from jax.experimental.pallas import tpu_sc as plsc

B, S, D = 4, 4096, 2048
K = 512
NC, NS = 2, 16
NW = NC * NS
ROWS_TOTAL = B * K
ROWS_PER_W = ROWS_TOTAL // NW
GC = 8
NCHUNK = ROWS_PER_W // GC


def _sumsq_body(key_ref, out_ref):
    x = key_ref[...]
    acc = x[:, :, 0:128] * x[:, :, 0:128]
    for c in range(1, D // 128):
        xc = x[:, :, c * 128:(c + 1) * 128]
        acc = acc + xc * xc
    u = acc[:, :, 0:8]
    for k in range(1, 16):
        u = u + acc[:, :, 8 * k:8 * k + 8]
    v1 = u[:, :, :4] + u[:, :, 4:]
    v2 = v1[:, :, :2] + v1[:, :, 2:]
    out_ref[...] = jnp.sqrt(v2[:, :, 0] + v2[:, :, 1])


S_BLK = 256


def _importance(key):
    return pl.pallas_call(
        _sumsq_body,
        grid=(S // S_BLK,),
        in_specs=[pl.BlockSpec((B, S_BLK, D), lambda i: (0, i, 0))],
        out_specs=pl.BlockSpec((B, S_BLK), lambda i: (0, i)),
        out_shape=jax.ShapeDtypeStruct((B, S), jnp.float32),
    )(key)


_PC = 512


def _select_body(imp_ref, out_ref):
    b = pl.program_id(0)
    v_row = imp_ref[0]
    bits = lax.bitcast_convert_type(v_row, jnp.int32)
    j_row = lax.broadcasted_iota(jnp.int32, (1, S), 1)

    t = jnp.zeros((1, 1), jnp.int32)
    for bit in range(30, -1, -1):
        cand = t | (1 << bit)
        cnt = jnp.sum((bits >= cand).astype(jnp.int32), axis=1, keepdims=True)
        t = jnp.where(cnt >= K, cand, t)
    strict = bits > t
    tie = bits == t
    m = jnp.sum(strict.astype(jnp.int32), axis=1, keepdims=True)
    need = K - m

    xm = jnp.zeros((1, 1), jnp.int32)
    for bit in range(11, -1, -1):
        cand = xm | (1 << bit)
        cnt = jnp.sum((tie & (j_row < cand)).astype(jnp.int32),
                      axis=1, keepdims=True)
        xm = jnp.where(cnt < need, cand, xm)
    sel_mask = strict | (tie & (j_row < (xm + 1)))
    candi = sel_mask.astype(jnp.int32)

    p = candi
    sh = 1
    while sh < S:
        shifted = jnp.concatenate(
            [jnp.zeros((1, sh), jnp.int32), p[:, :S - sh]], axis=1)
        p = p + shifted
        sh *= 2
    p_row = p - candi

    r_col = lax.broadcasted_iota(jnp.int32, (K, 1), 0)
    g = (p_row == r_col) & sel_mask
    bits_bc = jnp.broadcast_to(bits, (K, S))
    j_bc = jnp.broadcast_to(j_row, (K, S))
    bits_col = jnp.sum(jnp.where(g, bits_bc, 0), axis=1, keepdims=True)
    idx_col = jnp.sum(jnp.where(g, j_bc, 0), axis=1, keepdims=True)

    ii = lax.broadcasted_iota(jnp.int32, (K, K), 0)
    jj = lax.broadcasted_iota(jnp.int32, (K, K), 1)
    eye = ii == jj
    bits_row = jnp.sum(jnp.where(eye, jnp.broadcast_to(bits_col, (K, K)), 0),
                       axis=0, keepdims=True)
    idx_row = jnp.sum(jnp.where(eye, jnp.broadcast_to(idx_col, (K, K)), 0),
                      axis=0, keepdims=True)

    beats = (bits_row > bits_col) | ((bits_row == bits_col) &
                                     (idx_row < idx_col))
    rank_col = jnp.sum(beats.astype(jnp.int32), axis=1, keepdims=True)

    r_rowi = lax.broadcasted_iota(jnp.int32, (1, K), 1)
    sel = rank_col == r_rowi
    flat_col = jnp.broadcast_to(idx_col + b * S, (K, K))
    out_ref[0] = jnp.sum(jnp.where(sel, flat_col, 0), axis=0, keepdims=True)


def _topk_flat_idx(imp3):
    return pl.pallas_call(
        _select_body,
        grid=(B,),
        in_specs=[pl.BlockSpec((1, 1, S), lambda b: (b, 0, 0))],
        out_specs=pl.BlockSpec((1, 1, K), lambda b: (b, 0, 0)),
        out_shape=jax.ShapeDtypeStruct((B, 1, K), jnp.int32),
    )(imp3)


def _merged_body(key_ref, idx_ref, imp_s):
    i = pl.program_id(0)
    x = key_ref[...]
    acc = x[:, :, 0:128] * x[:, :, 0:128]
    for c in range(1, D // 128):
        xc = x[:, :, c * 128:(c + 1) * 128]
        acc = acc + xc * xc
    u = acc[:, :, 0:8]
    for k in range(1, 16):
        u = u + acc[:, :, 8 * k:8 * k + 8]
    v1 = u[:, :, :4] + u[:, :, 4:]
    v2 = v1[:, :, :2] + v1[:, :, 2:]
    imp_s[:, pl.ds(i * S_BLK, S_BLK)] = jnp.sqrt(v2[:, :, 0] + v2[:, :, 1])

    @pl.when(i == S // S_BLK - 1)
    def _epilogue():
        bits4 = lax.bitcast_convert_type(imp_s[...], jnp.int32)
        j_row4 = lax.broadcasted_iota(jnp.int32, (B, S), 1)

        t = jnp.zeros((B, 1), jnp.int32)
        for bit in range(30, -1, -1):
            cand = t | (1 << bit)
            cnt = jnp.sum((bits4 >= cand).astype(jnp.int32),
                          axis=1, keepdims=True)
            t = jnp.where(cnt >= K, cand, t)
        strict = bits4 > t
        tie = bits4 == t
        m = jnp.sum(strict.astype(jnp.int32), axis=1, keepdims=True)
        need = K - m

        xm = jnp.zeros((B, 1), jnp.int32)
        for bit in range(11, -1, -1):
            cand = xm | (1 << bit)
            cnt = jnp.sum((tie & (j_row4 < cand)).astype(jnp.int32),
                          axis=1, keepdims=True)
            xm = jnp.where(cnt < need, cand, xm)
        sel4 = strict | (tie & (j_row4 < (xm + 1)))
        candi = sel4.astype(jnp.int32)

        p = candi
        sh = 1
        while sh < S:
            shifted = jnp.concatenate(
                [jnp.zeros((B, sh), jnp.int32), p[:, :S - sh]], axis=1)
            p = p + shifted
            sh *= 2
        p4 = p - candi

        r_col = lax.broadcasted_iota(jnp.int32, (K, 1), 0)
        ii = lax.broadcasted_iota(jnp.int32, (K, K), 0)
        jj = lax.broadcasted_iota(jnp.int32, (K, K), 1)
        eye = ii == jj
        r_rowi = lax.broadcasted_iota(jnp.int32, (1, K), 1)
        j_row = j_row4[0:1, :]

        for b in range(B):
            bits = bits4[b:b + 1, :]
            g = (p4[b:b + 1, :] == r_col) & sel4[b:b + 1, :]
            bits_bc = jnp.broadcast_to(bits, (K, S))
            j_bc = jnp.broadcast_to(j_row, (K, S))
            bits_col = jnp.sum(jnp.where(g, bits_bc, 0), axis=1, keepdims=True)
            idx_col = jnp.sum(jnp.where(g, j_bc, 0), axis=1, keepdims=True)

            bits_row = jnp.sum(
                jnp.where(eye, jnp.broadcast_to(bits_col, (K, K)), 0),
                axis=0, keepdims=True)
            idx_row = jnp.sum(
                jnp.where(eye, jnp.broadcast_to(idx_col, (K, K)), 0),
                axis=0, keepdims=True)

            beats = (bits_row > bits_col) | ((bits_row == bits_col) &
                                             (idx_row < idx_col))
            rank_col = jnp.sum(beats.astype(jnp.int32), axis=1, keepdims=True)

            sel = rank_col == r_rowi
            flat_col = jnp.broadcast_to(idx_col + b * S, (K, K))
            idx_ref[b:b + 1, :] = jnp.sum(jnp.where(sel, flat_col, 0),
                                          axis=0, keepdims=True)


def _imp_topk(key):
    return pl.pallas_call(
        _merged_body,
        grid=(S // S_BLK,),
        in_specs=[pl.BlockSpec((B, S_BLK, D), lambda i: (0, i, 0))],
        out_specs=pl.BlockSpec((B, K), lambda i: (0, 0)),
        out_shape=jax.ShapeDtypeStruct((B, K), jnp.int32),
        scratch_shapes=[pltpu.VMEM((B, S), jnp.float32)],
    )(key)


def _gather_body(key_hbm, val_hbm, idx_hbm, out_hbm,
                 idx_all, od0, od1, ab0, ab1, sg0, sg1, ss0, ss1):
    cid = lax.axis_index("c")
    sid = lax.axis_index("s")
    wid = sid * NC + cid
    base = wid * ROWS_PER_W
    pltpu.sync_copy(idx_hbm.at[pl.ds(base, ROWS_PER_W)], idx_all)
    for t in range(ROWS_PER_W // 16):
        idx_all[pl.ds(t * 16, 16)] = jnp.clip(idx_all[pl.ds(t * 16, 16)],
                                              0, B * S - 1)

    ab = [ab0, ab1]
    od = [od0, od1]
    sg = [sg0, sg1]
    ss = [ss0, ss1]
    gops = [None, None]
    sops = [None, None]
    lanes = lax.iota(jnp.int32, 16)

    def start(ch):
        sl = ch % 2
        isrc = idx_all.at[pl.ds(ch * GC, GC)]
        gk = pltpu.async_copy(key_hbm.at[isrc], ab[sl].at[pl.ds(0, GC)],
                              sg[sl])
        gv = pltpu.async_copy(val_hbm.at[isrc], ab[sl].at[pl.ds(GC, GC)],
                              sg[sl])
        gops[sl] = (gk, gv)

    start(0)
    for ch in range(NCHUNK):
        sl = ch % 2
        if ch + 1 < NCHUNK:
            sl2 = (ch + 1) % 2
            if sops[sl2] is not None:
                sops[sl2].wait()
                sops[sl2] = None
            start(ch + 1)
        for op in gops[sl]:
            op.wait()
        od[sl][...] = 2 * (base + ch * GC + (lanes & 7)) + (lanes >> 3)
        sops[sl] = pltpu.async_copy(ab[sl], out_hbm.at[od[sl]], ss[sl])
    for sl in range(2):
        if sops[sl] is not None:
            sops[sl].wait()


@functools.cache
def _sc_gather():
    return pl.kernel(
        _gather_body,
        out_type=jax.ShapeDtypeStruct((2 * ROWS_TOTAL, D), jnp.float32),
        mesh=plsc.VectorSubcoreMesh(core_axis_name="c", subcore_axis_name="s"),
        scratch_types=[
            pltpu.VMEM((ROWS_PER_W,), jnp.int32),
            pltpu.VMEM((16,), jnp.int32),
            pltpu.VMEM((16,), jnp.int32),
            pltpu.VMEM((2 * GC, D), jnp.float32),
            pltpu.VMEM((2 * GC, D), jnp.float32),
            pltpu.SemaphoreType.DMA,
            pltpu.SemaphoreType.DMA,
            pltpu.SemaphoreType.DMA,
            pltpu.SemaphoreType.DMA,
        ],
    )


def kernel(key, value):
    idx = _imp_topk(key)
    out2 = _sc_gather()(key.reshape(B * S, D), value.reshape(B * S, D),
                        idx.reshape(ROWS_TOTAL))
    return out2.reshape(B, K, 2 * D)


# speedup vs baseline: 1.4018x; 1.0042x over previous
import functools

import jax
import jax.numpy as jnp
from jax import lax
from jax.experimental import pallas as pl
from jax.experimental.pallas import tpu as pltpu
from jax.experimental.pallas import tpu_sc as plsc

B, S, D = 4, 4096, 2048
K = 512
NC, NS = 2, 16
NW = NC * NS
ROWS_TOTAL = B * K
ROWS_PER_W = ROWS_TOTAL // NW
GC = 8
NCHUNK = ROWS_PER_W // GC


def _sumsq_body(key_ref, out_ref):
    x = key_ref[...]
    acc = x[:, :, 0:128] * x[:, :, 0:128]
    for c in range(1, D // 128):
        xc = x[:, :, c * 128:(c + 1) * 128]
        acc = acc + xc * xc
    u = acc[:, :, 0:8]
    for k in range(1, 16):
        u = u + acc[:, :, 8 * k:8 * k + 8]
    v1 = u[:, :, :4] + u[:, :, 4:]
    v2 = v1[:, :, :2] + v1[:, :, 2:]
    out_ref[...] = jnp.sqrt(v2[:, :, 0] + v2[:, :, 1])


S_BLK = 256


def _importance(key):
    return pl.pallas_call(
        _sumsq_body,
        grid=(S // S_BLK,),
        in_specs=[pl.BlockSpec((B, S_BLK, D), lambda i: (0, i, 0))],
        out_specs=pl.BlockSpec((B, S_BLK), lambda i: (0, i)),
        out_shape=jax.ShapeDtypeStruct((B, S), jnp.float32),
    )(key)


_PC = 512


def _select_body(imp_ref, out_ref):
    b = pl.program_id(0)
    v_row = imp_ref[0]
    bits = lax.bitcast_convert_type(v_row, jnp.int32)
    j_row = lax.broadcasted_iota(jnp.int32, (1, S), 1)

    t = jnp.zeros((1, 1), jnp.int32)
    for bit in range(30, -1, -1):
        cand = t | (1 << bit)
        cnt = jnp.sum((bits >= cand).astype(jnp.int32), axis=1, keepdims=True)
        t = jnp.where(cnt >= K, cand, t)
    strict = bits > t
    tie = bits == t
    m = jnp.sum(strict.astype(jnp.int32), axis=1, keepdims=True)
    need = K - m

    xm = jnp.zeros((1, 1), jnp.int32)
    for bit in range(11, -1, -1):
        cand = xm | (1 << bit)
        cnt = jnp.sum((tie & (j_row < cand)).astype(jnp.int32),
                      axis=1, keepdims=True)
        xm = jnp.where(cnt < need, cand, xm)
    sel_mask = strict | (tie & (j_row < (xm + 1)))
    candi = sel_mask.astype(jnp.int32)

    p = candi
    sh = 1
    while sh < S:
        shifted = jnp.concatenate(
            [jnp.zeros((1, sh), jnp.int32), p[:, :S - sh]], axis=1)
        p = p + shifted
        sh *= 2
    p_row = p - candi

    r_col = lax.broadcasted_iota(jnp.int32, (K, 1), 0)
    g = (p_row == r_col) & sel_mask
    bits_bc = jnp.broadcast_to(bits, (K, S))
    j_bc = jnp.broadcast_to(j_row, (K, S))
    bits_col = jnp.sum(jnp.where(g, bits_bc, 0), axis=1, keepdims=True)
    idx_col = jnp.sum(jnp.where(g, j_bc, 0), axis=1, keepdims=True)

    ii = lax.broadcasted_iota(jnp.int32, (K, K), 0)
    jj = lax.broadcasted_iota(jnp.int32, (K, K), 1)
    eye = ii == jj
    bits_row = jnp.sum(jnp.where(eye, jnp.broadcast_to(bits_col, (K, K)), 0),
                       axis=0, keepdims=True)
    idx_row = jnp.sum(jnp.where(eye, jnp.broadcast_to(idx_col, (K, K)), 0),
                      axis=0, keepdims=True)

    beats = (bits_row > bits_col) | ((bits_row == bits_col) &
                                     (idx_row < idx_col))
    rank_col = jnp.sum(beats.astype(jnp.int32), axis=1, keepdims=True)

    r_rowi = lax.broadcasted_iota(jnp.int32, (1, K), 1)
    sel = rank_col == r_rowi
    flat_col = jnp.broadcast_to(idx_col + b * S, (K, K))
    out_ref[0] = jnp.sum(jnp.where(sel, flat_col, 0), axis=0, keepdims=True)


def _topk_flat_idx(imp3):
    return pl.pallas_call(
        _select_body,
        grid=(B,),
        in_specs=[pl.BlockSpec((1, 1, S), lambda b: (b, 0, 0))],
        out_specs=pl.BlockSpec((1, 1, K), lambda b: (b, 0, 0)),
        out_shape=jax.ShapeDtypeStruct((B, 1, K), jnp.int32),
    )(imp3)


def _merged_body(key_ref, idx_ref, imp_s):
    i = pl.program_id(0)
    x = key_ref[...]
    acc = x[:, :, 0:128] * x[:, :, 0:128]
    for c in range(1, D // 128):
        xc = x[:, :, c * 128:(c + 1) * 128]
        acc = acc + xc * xc
    u = acc[:, :, 0:8]
    for k in range(1, 16):
        u = u + acc[:, :, 8 * k:8 * k + 8]
    v1 = u[:, :, :4] + u[:, :, 4:]
    v2 = v1[:, :, :2] + v1[:, :, 2:]
    imp_s[:, pl.ds(i * S_BLK, S_BLK)] = jnp.sqrt(v2[:, :, 0] + v2[:, :, 1])

    @pl.when(i == S // S_BLK - 1)
    def _epilogue():
        bits4 = lax.bitcast_convert_type(imp_s[...], jnp.int32)
        j_row4 = lax.broadcasted_iota(jnp.int32, (B, S), 1)

        t = jnp.zeros((B, 1), jnp.int32)
        for bit in range(30, -1, -1):
            cand = t | (1 << bit)
            cnt = jnp.sum((bits4 >= cand).astype(jnp.int32),
                          axis=1, keepdims=True)
            t = jnp.where(cnt >= K, cand, t)
        strict = bits4 > t
        tie = bits4 == t
        m = jnp.sum(strict.astype(jnp.int32), axis=1, keepdims=True)
        need = K - m

        xm = jnp.zeros((B, 1), jnp.int32)
        for bit in range(11, -1, -1):
            cand = xm | (1 << bit)
            cnt = jnp.sum((tie & (j_row4 < cand)).astype(jnp.int32),
                          axis=1, keepdims=True)
            xm = jnp.where(cnt < need, cand, xm)
        sel4 = strict | (tie & (j_row4 < (xm + 1)))
        candi = sel4.astype(jnp.int32)

        p = candi
        sh = 1
        while sh < S:
            shifted = jnp.concatenate(
                [jnp.zeros((B, sh), jnp.int32), p[:, :S - sh]], axis=1)
            p = p + shifted
            sh *= 2
        p4 = p - candi

        r_col = lax.broadcasted_iota(jnp.int32, (K, 1), 0)
        ii = lax.broadcasted_iota(jnp.int32, (K, K), 0)
        jj = lax.broadcasted_iota(jnp.int32, (K, K), 1)
        eye = ii == jj
        r_rowi = lax.broadcasted_iota(jnp.int32, (1, K), 1)
        j_row = j_row4[0:1, :]

        for b in range(B):
            bits = bits4[b:b + 1, :]
            g = (p4[b:b + 1, :] == r_col) & sel4[b:b + 1, :]
            bits_bc = jnp.broadcast_to(bits, (K, S))
            j_bc = jnp.broadcast_to(j_row, (K, S))
            bits_col = jnp.sum(jnp.where(g, bits_bc, 0), axis=1, keepdims=True)
            idx_col = jnp.sum(jnp.where(g, j_bc, 0), axis=1, keepdims=True)

            bits_row = jnp.sum(
                jnp.where(eye, jnp.broadcast_to(bits_col, (K, K)), 0),
                axis=0, keepdims=True)
            idx_row = jnp.sum(
                jnp.where(eye, jnp.broadcast_to(idx_col, (K, K)), 0),
                axis=0, keepdims=True)

            beats = (bits_row > bits_col) | ((bits_row == bits_col) &
                                             (idx_row < idx_col))
            rank_col = jnp.sum(beats.astype(jnp.int32), axis=1, keepdims=True)

            sel = rank_col == r_rowi
            flat_col = jnp.broadcast_to(idx_col + b * S, (K, K))
            row = jnp.sum(jnp.where(sel, flat_col, 0), axis=0, keepdims=True)
            idx_ref[pl.ds(b * K, K)] = row.reshape(K)


def _imp_topk(key):
    return pl.pallas_call(
        _merged_body,
        grid=(S // S_BLK,),
        in_specs=[pl.BlockSpec((B, S_BLK, D), lambda i: (0, i, 0))],
        out_specs=pl.BlockSpec((ROWS_TOTAL,), lambda i: (0,)),
        out_shape=jax.ShapeDtypeStruct((ROWS_TOTAL,), jnp.int32),
        scratch_shapes=[pltpu.VMEM((B, S), jnp.float32)],
    )(key)


def _gather_body(key_hbm, val_hbm, idx_hbm, out_hbm,
                 idx_all, od0, od1, ab0, ab1, sg0, sg1, ss0, ss1):
    cid = lax.axis_index("c")
    sid = lax.axis_index("s")
    wid = sid * NC + cid
    base = wid * ROWS_PER_W
    pltpu.sync_copy(idx_hbm.at[pl.ds(base, ROWS_PER_W)], idx_all)
    for t in range(ROWS_PER_W // 16):
        idx_all[pl.ds(t * 16, 16)] = jnp.clip(idx_all[pl.ds(t * 16, 16)],
                                              0, B * S - 1)

    ab = [ab0, ab1]
    od = [od0, od1]
    sg = [sg0, sg1]
    ss = [ss0, ss1]
    gops = [None, None]
    sops = [None, None]
    lanes = lax.iota(jnp.int32, 16)

    def start(ch):
        sl = ch % 2
        isrc = idx_all.at[pl.ds(ch * GC, GC)]
        gk = pltpu.async_copy(key_hbm.at[isrc], ab[sl].at[pl.ds(0, GC)],
                              sg[sl])
        gv = pltpu.async_copy(val_hbm.at[isrc], ab[sl].at[pl.ds(GC, GC)],
                              sg[sl])
        gops[sl] = (gk, gv)

    start(0)
    for ch in range(NCHUNK):
        sl = ch % 2
        if ch + 1 < NCHUNK:
            sl2 = (ch + 1) % 2
            if sops[sl2] is not None:
                sops[sl2].wait()
                sops[sl2] = None
            start(ch + 1)
        for op in gops[sl]:
            op.wait()
        od[sl][...] = 2 * (base + ch * GC + (lanes & 7)) + (lanes >> 3)
        sops[sl] = pltpu.async_copy(ab[sl], out_hbm.at[od[sl]], ss[sl])
    for sl in range(2):
        if sops[sl] is not None:
            sops[sl].wait()


@functools.cache
def _sc_gather():
    return pl.kernel(
        _gather_body,
        out_type=jax.ShapeDtypeStruct((2 * ROWS_TOTAL, D), jnp.float32),
        mesh=plsc.VectorSubcoreMesh(core_axis_name="c", subcore_axis_name="s"),
        scratch_types=[
            pltpu.VMEM((ROWS_PER_W,), jnp.int32),
            pltpu.VMEM((16,), jnp.int32),
            pltpu.VMEM((16,), jnp.int32),
            pltpu.VMEM((2 * GC, D), jnp.float32),
            pltpu.VMEM((2 * GC, D), jnp.float32),
            pltpu.SemaphoreType.DMA,
            pltpu.SemaphoreType.DMA,
            pltpu.SemaphoreType.DMA,
            pltpu.SemaphoreType.DMA,
        ],
    )


def kernel(key, value):
    idx = _imp_topk(key)
    out2 = _sc_gather()(key.reshape(B * S, D), value.reshape(B * S, D), idx)
    return out2.reshape(B, K, 2 * D)
